# Initial kernel scaffold; baseline (speedup 1.0000x reference)
import jax
import jax.numpy as jnp
from jax.experimental import pallas as pl


def kernel(node_num_list, visual_feat, spatial_feat, We, be, Wa, ba, Wn, bn, Wp, bp):
    raise NotImplementedError("write your pallas kernel here")


# speedup vs baseline: 30.2328x; 30.2328x over previous
import jax
import jax.numpy as jnp
from jax.experimental import pallas as pl


def _gdm_kernel(vf_ref, sfp_ref, We1_ref, We2_ref, We3_ref, be_ref, Wa_ref,
                ba_ref, Wn1_ref, Wn2_ref, bn_ref, Wp1_ref, Wp2_ref, Wp3_ref,
                bp_ref, out_ref):
    k, dn = vf_ref.shape
    vf = vf_ref[...]
    sfp = sfp_ref[...]
    u = jnp.dot(vf, We1_ref[...], preferred_element_type=jnp.float32) + be_ref[...]
    v = jnp.dot(vf, We3_ref[...], preferred_element_type=jnp.float32)
    sw = jnp.dot(sfp, We2_ref[...], preferred_element_type=jnp.float32)
    sw3 = sw.reshape(k, k - 1, dn)
    zc = jnp.zeros((k, 1, dn), jnp.float32)
    x = jnp.concatenate([sw3, zc], axis=1)
    y = jnp.concatenate([zc, sw3], axis=1)
    s3 = jax.lax.broadcasted_iota(jnp.int32, (k, k, 1), 0)
    d3 = jax.lax.broadcasted_iota(jnp.int32, (k, k, 1), 1)
    swd = jnp.where(d3 < s3, x, y)
    ef = jax.nn.relu(swd + u[:, None, :] + v[None, :, :])
    a = jax.nn.relu(jnp.sum(ef * Wa_ref[...].reshape(1, 1, dn), axis=-1)
                    + ba_ref[...])
    s2 = jax.lax.broadcasted_iota(jnp.int32, (k, k), 0)
    d2 = jax.lax.broadcasted_iota(jnp.int32, (k, k), 1)
    a = jnp.where(s2 == d2, jnp.float32(-1e30), a)
    m = jnp.max(a, axis=0, keepdims=True)
    ex = jnp.exp(a - m)
    den = jnp.sum(ex, axis=0, keepdims=True)
    alpha = ex / den
    msg = vf[:, None, :] + ef
    z = jnp.sum(alpha[:, :, None] * msg, axis=0)
    nn = jax.nn.relu(jnp.dot(vf, Wn1_ref[...], preferred_element_type=jnp.float32)
                     + jnp.dot(z, Wn2_ref[...], preferred_element_type=jnp.float32)
                     + bn_ref[...])
    pred = (jnp.dot(nn[0:1, :], Wp1_ref[...], preferred_element_type=jnp.float32)
            + jnp.dot(sfp[0:k - 1, :], Wp2_ref[...], preferred_element_type=jnp.float32)
            + jnp.dot(nn[1:k, :], Wp3_ref[...], preferred_element_type=jnp.float32)
            + bp_ref[...])
    out_ref[...] = pred


def kernel(node_num_list, visual_feat, spatial_feat, We, be, Wa, ba, Wn, bn,
           Wp, bp):
    b = node_num_list.shape[0]
    n, dn = visual_feat.shape
    k = n // b
    ds = spatial_feat.shape[1]
    dp = Wp.shape[1]
    epg = k * (k - 1)
    We1, We2, We3 = We[:dn], We[dn:dn + ds], We[dn + ds:]
    Wn1, Wn2 = Wn[:dn], Wn[dn:]
    Wp1, Wp2, Wp3 = Wp[:dn], Wp[dn:dn + ds], Wp[dn + ds:]

    def const(*shape):
        return pl.BlockSpec(shape, lambda i: tuple(0 for _ in shape))

    out = pl.pallas_call(
        _gdm_kernel,
        grid=(b,),
        in_specs=[
            pl.BlockSpec((k, dn), lambda i: (i, 0)),
            pl.BlockSpec((epg, ds), lambda i: (i, 0)),
            const(dn, dn), const(ds, dn), const(dn, dn), const(1, dn),
            const(1, dn), const(1, 1),
            const(dn, dn), const(dn, dn), const(1, dn),
            const(dn, dp), const(ds, dp), const(dn, dp), const(1, dp),
        ],
        out_specs=pl.BlockSpec((None, k - 1, dp), lambda i: (i, 0, 0)),
        out_shape=jax.ShapeDtypeStruct((b, k - 1, dp), jnp.float32),
    )(visual_feat, spatial_feat, We1, We2, We3, be.reshape(1, dn),
      Wa.reshape(1, dn), ba.reshape(1, 1), Wn1, Wn2, bn.reshape(1, dn),
      Wp1, Wp2, Wp3, bp.reshape(1, dp))
    return out.reshape(b * (k - 1), dp)
